# baseline (device time: 230713 ns/iter reference)
import jax
import jax.numpy as jnp
from jax import lax
from jax.experimental import pallas as pl
from jax.experimental.pallas import tpu as pltpu

N_DEV = 32
E_LOC = 4
N_EXP = 128
CAPD = 112
DW = 640
BLK = 8
N_BLK = N_DEV // BLK


def kernel(x, router_W, route_idx, expert_W, shared_W):
    n_tok, d = x.shape
    h = shared_W.shape[1]

    e = route_idx[:, 0]
    dest = e // E_LOC
    ohd = (dest[:, None] == jnp.arange(N_DEV)[None, :]).astype(jnp.int32)
    pos = jnp.sum((jnp.cumsum(ohd, axis=0) - ohd) * ohd, axis=1)
    k = jnp.where(pos < CAPD, dest * CAPD + pos, -1).astype(jnp.int32)

    k_row = k[None, :]
    k_col = k[:, None]
    e_col = e[:, None].astype(jnp.int32)
    x_bf = x.astype(jnp.bfloat16)
    rW = router_W.astype(jnp.bfloat16)
    sW = shared_W.astype(jnp.bfloat16)
    eW = expert_W.astype(jnp.bfloat16)

    def body(x_ref, rW_ref, ecol_ref, krow_ref, kcol_ref, sW_ref, eW_ref,
             out_ref, disp_ref, r_ref, y_ref, z_ref,
             send1, recv1, send2, recv2, cp_sem1, cp_sem2):
        me = lax.axis_index("i")
        xv = x_ref[...]
        ecol = ecol_ref[...]
        krow = krow_ref[...]

        sc = jnp.dot(xv, rW_ref[...], preferred_element_type=jnp.float32)
        sc = sc - jnp.max(sc, axis=1, keepdims=True)
        pr = jnp.exp(sc)
        pr = pr / jnp.sum(pr, axis=1, keepdims=True)
        oh_e = (ecol == lax.broadcasted_iota(
            jnp.int32, (n_tok, N_EXP), 1)).astype(jnp.float32)
        p = jnp.sum(pr * oh_e, axis=1, keepdims=True)
        xs = xv * p.astype(jnp.bfloat16)
        lev_col = (ecol % E_LOC).astype(jnp.bfloat16)

        def build_send(t, carry):
            dd = lax.rem(me + 1 + t, N_DEV)
            oh = (krow == lax.broadcasted_iota(jnp.int32, (CAPD, n_tok), 0)
                  + dd * CAPD).astype(jnp.bfloat16)
            chunk = jnp.dot(oh, xs, preferred_element_type=jnp.float32)
            slot_le = jnp.dot(oh, lev_col,
                              preferred_element_type=jnp.float32)
            disp_ref[pl.ds(dd, 1), :, pl.ds(0, d)] = (
                chunk.astype(jnp.bfloat16).reshape(1, CAPD, d))
            disp_ref[pl.ds(dd, 1), :, pl.ds(d, DW - d)] = jnp.broadcast_to(
                slot_le.astype(jnp.bfloat16).reshape(1, CAPD, 1),
                (1, CAPD, DW - d))

            @pl.when(dd != me)
            def _():
                pltpu.make_async_remote_copy(
                    src_ref=disp_ref.at[dd],
                    dst_ref=r_ref.at[me],
                    send_sem=send1.at[dd],
                    recv_sem=recv1.at[me],
                    device_id=dd,
                    device_id_type=pl.DeviceIdType.LOGICAL,
                ).start()

            @pl.when(dd == me)
            def _():
                cp = pltpu.make_async_copy(
                    disp_ref.at[dd], r_ref.at[dd], cp_sem1)
                cp.start()
                cp.wait()
            return carry
        lax.fori_loop(0, N_DEV, build_send, 0)

        sh = jnp.dot(xv, sW_ref[...], preferred_element_type=jnp.float32)
        out_ref[...] = sh.astype(jnp.bfloat16)

        for b in range(N_BLK):
            def wait_disp(j, carry):
                s = b * BLK + j

                @pl.when(s != me)
                def _():
                    pltpu.make_async_remote_copy(
                        src_ref=disp_ref.at[0],
                        dst_ref=r_ref.at[s],
                        send_sem=send1.at[s],
                        recv_sem=recv1.at[s],
                        device_id=me,
                        device_id_type=pl.DeviceIdType.LOGICAL,
                    ).wait_recv()
                return carry
            lax.fori_loop(0, BLK, wait_disp, 0)

            flat = r_ref[pl.ds(b * BLK, BLK), :, :].reshape(BLK * CAPD, DW)
            xin = flat[:, :d]
            lev = flat[:, d:d + 1]
            yv = jnp.zeros((BLK * CAPD, h), jnp.float32)
            for lei in range(E_LOC):
                mask = (lev == lei).astype(jnp.bfloat16)
                yv = yv + jnp.dot(xin * mask, eW_ref[lei],
                                  preferred_element_type=jnp.float32)
            y_ref[pl.ds(b * BLK, BLK), :, :] = (
                yv.astype(jnp.bfloat16).reshape(BLK, CAPD, h))

            def send_ret(j, carry):
                s = b * BLK + j

                @pl.when(s != me)
                def _():
                    pltpu.make_async_remote_copy(
                        src_ref=y_ref.at[s],
                        dst_ref=z_ref.at[me],
                        send_sem=send2.at[s],
                        recv_sem=recv2.at[me],
                        device_id=s,
                        device_id_type=pl.DeviceIdType.LOGICAL,
                    ).start()

                @pl.when(s == me)
                def _():
                    cp = pltpu.make_async_copy(
                        y_ref.at[s], z_ref.at[s], cp_sem2)
                    cp.start()
                    cp.wait()
                return carry
            lax.fori_loop(0, BLK, send_ret, 0)

        kcol = kcol_ref[...]
        for b in range(N_BLK):
            def wait_ret(j, carry):
                s = b * BLK + j

                @pl.when(s != me)
                def _():
                    pltpu.make_async_remote_copy(
                        src_ref=y_ref.at[0],
                        dst_ref=z_ref.at[s],
                        send_sem=send2.at[s],
                        recv_sem=recv2.at[s],
                        device_id=me,
                        device_id_type=pl.DeviceIdType.LOGICAL,
                    ).wait_recv()
                return carry
            lax.fori_loop(0, BLK, wait_ret, 0)

            oh = (kcol == lax.broadcasted_iota(
                jnp.int32, (n_tok, BLK * CAPD), 1)
                + b * BLK * CAPD).astype(jnp.bfloat16)
            zb = z_ref[pl.ds(b * BLK, BLK)].reshape(BLK * CAPD, h)
            out_ref[...] = out_ref[...] + jnp.dot(
                oh, zb, preferred_element_type=jnp.float32).astype(jnp.bfloat16)

        def wait_sends(t, carry):
            dd = lax.rem(me + 1 + t, N_DEV)
            pltpu.make_async_remote_copy(
                src_ref=disp_ref.at[0],
                dst_ref=r_ref.at[0],
                send_sem=send1.at[dd],
                recv_sem=recv1.at[0],
                device_id=me,
                device_id_type=pl.DeviceIdType.LOGICAL,
            ).wait_send()
            pltpu.make_async_remote_copy(
                src_ref=y_ref.at[0],
                dst_ref=z_ref.at[0],
                send_sem=send2.at[dd],
                recv_sem=recv2.at[0],
                device_id=me,
                device_id_type=pl.DeviceIdType.LOGICAL,
            ).wait_send()
            return carry
        lax.fori_loop(0, N_DEV - 1, wait_sends, 0)

    out_bf = pl.pallas_call(
        body,
        out_shape=jax.ShapeDtypeStruct((n_tok, h), jnp.bfloat16),
        in_specs=[pl.BlockSpec(memory_space=pltpu.VMEM)] * 7,
        out_specs=pl.BlockSpec(memory_space=pltpu.VMEM),
        scratch_shapes=[
            pltpu.VMEM((N_DEV, CAPD, DW), jnp.bfloat16),
            pltpu.VMEM((N_DEV, CAPD, DW), jnp.bfloat16),
            pltpu.VMEM((N_DEV, CAPD, h), jnp.bfloat16),
            pltpu.VMEM((N_DEV, CAPD, h), jnp.bfloat16),
            pltpu.SemaphoreType.DMA((N_DEV,)),
            pltpu.SemaphoreType.DMA((N_DEV,)),
            pltpu.SemaphoreType.DMA((N_DEV,)),
            pltpu.SemaphoreType.DMA((N_DEV,)),
            pltpu.SemaphoreType.DMA,
            pltpu.SemaphoreType.DMA,
        ],
        compiler_params=pltpu.CompilerParams(
            has_side_effects=True,
            vmem_limit_bytes=52 * 1024 * 1024,
        ),
    )(x_bf, rW, e_col, k_row, k_col, sW, eW)

    return out_bf.astype(jnp.float32)
